# baseline (speedup 1.0000x reference)
import functools

import jax
import jax.numpy as jnp
from jax import lax
from jax.experimental import pallas as pl
from jax.experimental.pallas import tpu as pltpu
from jax.experimental.pallas import tpu_sc as plsc

B, N, F = 256, 128, 9
H = 768
NB1 = N + 1
NW = 32
BPW = B // NW
NODES_PW = BPW * N
C = 8
ROWS = C * F
CHUNKS_PB = N // C
CHUNKS = NODES_PW // C
LANES = 16
G = H // (2 * LANES)


def _body(idx_hbm, table_hbm, token_hbm, out_hbm,
          idx_v, rows0, rows1, acc_v, tok_v, sem0, sem1, osem):
    cid = lax.axis_index("c")
    sid = lax.axis_index("s")
    wid = sid * 2 + cid
    node0 = wid * NODES_PW

    pltpu.sync_copy(idx_hbm.at[pl.ds(node0 * F, NODES_PW * F)], idx_v)

    pltpu.sync_copy(token_hbm, tok_v)
    for b in range(BPW):
        row = (wid * BPW + b) * NB1
        pltpu.sync_copy(tok_v, out_hbm.at[pl.ds(pl.multiple_of(row * H, 8), H)])

    def gather(k, buf, sem):
        start = pl.multiple_of(k * ROWS, 8)
        return pltpu.make_async_copy(
            table_hbm.at[idx_v.at[pl.ds(start, ROWS)]], buf, sem)

    def out_copy(k):
        b = k // CHUNKS_PB
        n = (k % CHUNKS_PB) * C
        row = (wid * BPW + b) * NB1 + 1 + n
        return pltpu.make_async_copy(
            acc_v, out_hbm.at[pl.ds(pl.multiple_of(row * H, 8), C * H)], osem)

    def process(k, buf):
        @pl.when(k > 0)
        def _():
            out_copy(k).wait()

        def g_body(g, carry):
            off = g * (2 * LANES)
            for i in range(C):
                base = i * F
                v = buf[base, pl.ds(off, 2 * LANES)]
                sa, sb = plsc.unpack(v, format=plsc.PackFormat.INTERLEAVED)
                for f in range(1, F):
                    v = buf[base + f, pl.ds(off, 2 * LANES)]
                    a, b2 = plsc.unpack(v, format=plsc.PackFormat.INTERLEAVED)
                    sa = sa + a
                    sb = sb + b2
                acc_v[pl.ds(i * H + off, LANES)] = sa
                acc_v[pl.ds(i * H + off + LANES, LANES)] = sb
            return carry

        lax.fori_loop(0, G, g_body, 0)
        out_copy(k).start()

    gather(0, rows0, sem0).start()

    def pair_body(j, carry):
        k0 = 2 * j
        k1 = k0 + 1
        gather(k1, rows1, sem1).start()
        gather(k0, rows0, sem0).wait()
        process(k0, rows0)

        @pl.when(k1 + 1 < CHUNKS)
        def _():
            gather(k1 + 1, rows0, sem0).start()

        gather(k1, rows1, sem1).wait()
        process(k1, rows1)
        return carry

    lax.fori_loop(0, CHUNKS // 2, pair_body, 0)
    out_copy(CHUNKS - 1).wait()


_sc_call = pl.kernel(
    _body,
    out_type=jax.ShapeDtypeStruct((B * NB1 * H,), jnp.float32),
    mesh=plsc.VectorSubcoreMesh(core_axis_name="c", subcore_axis_name="s"),
    compiler_params=pltpu.CompilerParams(
        use_tc_tiling_on_sc=False, needs_layout_passes=False),
    scratch_types=[
        pltpu.VMEM((NODES_PW * F,), jnp.int32),
        pltpu.VMEM((ROWS, H), jnp.bfloat16),
        pltpu.VMEM((ROWS, H), jnp.bfloat16),
        pltpu.VMEM((C * H,), jnp.float32),
        pltpu.VMEM((H,), jnp.float32),
        pltpu.SemaphoreType.DMA,
        pltpu.SemaphoreType.DMA,
        pltpu.SemaphoreType.DMA,
    ],
)


@jax.jit
def kernel(input_nodes, atom_encoder_weight, graph_token_weight):
    idx_flat = input_nodes.reshape(B * N * F)
    tb = atom_encoder_weight.astype(jnp.bfloat16)
    tb = tb.reshape(-1, G, 2, LANES).transpose(0, 1, 3, 2).reshape(-1, H)
    out = _sc_call(idx_flat, tb, graph_token_weight.reshape(H))
    return out.reshape(B, NB1, H)


# speedup vs baseline: 1.6526x; 1.6526x over previous
import functools

import jax
import jax.numpy as jnp
from jax import lax
from jax.experimental import pallas as pl
from jax.experimental.pallas import tpu as pltpu
from jax.experimental.pallas import tpu_sc as plsc

B, N, F = 256, 128, 9
H = 768
NB1 = N + 1
NW = 32
BPW = B // NW
NODES_PW = BPW * N
C = 8
ROWS = C * F
CHUNKS_PB = N // C
CHUNKS = NODES_PW // C
LANES = 16
G = H // (2 * LANES)


def _body(idx_hbm, table_hbm, token_hbm, out_hbm,
          idx_v, rows0, rows1, acc_v, tok_v, sem0, sem1, osem):
    cid = lax.axis_index("c")
    sid = lax.axis_index("s")
    wid = sid * 2 + cid
    node0 = wid * NODES_PW

    pltpu.sync_copy(idx_hbm.at[pl.ds(node0 * F, NODES_PW * F)], idx_v)

    pltpu.sync_copy(token_hbm, tok_v)
    for b in range(BPW):
        row = (wid * BPW + b) * NB1
        pltpu.sync_copy(tok_v, out_hbm.at[pl.ds(pl.multiple_of(row * H, 8), H)])

    def gather(k, buf, sem):
        start = pl.multiple_of(k * ROWS, 8)
        return pltpu.make_async_copy(
            table_hbm.at[idx_v.at[pl.ds(start, ROWS)]], buf, sem)

    def out_copy(k):
        b = k // CHUNKS_PB
        n = (k % CHUNKS_PB) * C
        row = (wid * BPW + b) * NB1 + 1 + n
        return pltpu.make_async_copy(
            acc_v, out_hbm.at[pl.ds(pl.multiple_of(row * H, 8), C * H)], osem)

    def process(k, buf):
        @pl.when(k > 0)
        def _():
            out_copy(k).wait()

        evens = 2 * lax.iota(jnp.int32, LANES)

        def g_body(g, carry):
            off = g * (2 * LANES)
            for i in range(C):
                base = i * F
                s = buf[base, pl.ds(off, 2 * LANES)]
                for f in range(1, F):
                    s = s + buf[base + f, pl.ds(off, 2 * LANES)]
                sa, sb = plsc.unpack(s, format=plsc.PackFormat.INTERLEAVED)
                pos = evens + (i * H + off)
                plsc.store_scatter(acc_v, [pos], sa)
                plsc.store_scatter(acc_v, [pos + 1], sb)
            return carry

        lax.fori_loop(0, G, g_body, 0)
        out_copy(k).start()

    gather(0, rows0, sem0).start()

    def pair_body(j, carry):
        k0 = 2 * j
        k1 = k0 + 1
        gather(k1, rows1, sem1).start()
        gather(k0, rows0, sem0).wait()
        process(k0, rows0)

        @pl.when(k1 + 1 < CHUNKS)
        def _():
            gather(k1 + 1, rows0, sem0).start()

        gather(k1, rows1, sem1).wait()
        process(k1, rows1)
        return carry

    lax.fori_loop(0, CHUNKS // 2, pair_body, 0)
    out_copy(CHUNKS - 1).wait()


_sc_call = pl.kernel(
    _body,
    out_type=jax.ShapeDtypeStruct((B * NB1 * H,), jnp.float32),
    mesh=plsc.VectorSubcoreMesh(core_axis_name="c", subcore_axis_name="s"),
    compiler_params=pltpu.CompilerParams(
        use_tc_tiling_on_sc=False, needs_layout_passes=False),
    scratch_types=[
        pltpu.VMEM((NODES_PW * F,), jnp.int32),
        pltpu.VMEM((ROWS, H), jnp.bfloat16),
        pltpu.VMEM((ROWS, H), jnp.bfloat16),
        pltpu.VMEM((C * H,), jnp.float32),
        pltpu.VMEM((H,), jnp.float32),
        pltpu.SemaphoreType.DMA,
        pltpu.SemaphoreType.DMA,
        pltpu.SemaphoreType.DMA,
    ],
)


@jax.jit
def kernel(input_nodes, atom_encoder_weight, graph_token_weight):
    idx_flat = input_nodes.reshape(B * N * F)
    tb = atom_encoder_weight.astype(jnp.bfloat16)
    out = _sc_call(idx_flat, tb, graph_token_weight.reshape(H))
    return out.reshape(B, NB1, H)


# speedup vs baseline: 2.4032x; 1.4542x over previous
import functools

import jax
import jax.numpy as jnp
from jax import lax
from jax.experimental import pallas as pl
from jax.experimental.pallas import tpu as pltpu
from jax.experimental.pallas import tpu_sc as plsc

B, N, F = 256, 128, 9
H = 768
NB1 = N + 1
NW = 32
BPW = B // NW
NODES_PW = BPW * N
C = 8
ROWS = C * F
CHUNKS_PB = N // C
CHUNKS = NODES_PW // C
LANES = 16
HV = H // LANES


def _body(idx_hbm, table_hbm, token_hbm, out_hbm,
          idx_v, rows0, rows1, acc_v, tok_v, sem0, sem1, osem):
    cid = lax.axis_index("c")
    sid = lax.axis_index("s")
    wid = sid * 2 + cid
    node0 = wid * NODES_PW

    pltpu.sync_copy(idx_hbm.at[pl.ds(node0 * F, NODES_PW * F)], idx_v)

    pltpu.sync_copy(token_hbm, tok_v)
    for b in range(BPW):
        row = (wid * BPW + b) * NB1
        pltpu.sync_copy(tok_v, out_hbm.at[pl.ds(pl.multiple_of(row * H, 8), H)])

    def gather(k, buf, sem):
        start = pl.multiple_of(k * ROWS, 8)
        return pltpu.make_async_copy(
            table_hbm.at[idx_v.at[pl.ds(start, ROWS)]], buf, sem)

    def out_copy(k):
        b = k // CHUNKS_PB
        n = (k % CHUNKS_PB) * C
        row = (wid * BPW + b) * NB1 + 1 + n
        return pltpu.make_async_copy(
            acc_v, out_hbm.at[pl.ds(pl.multiple_of(row * H, 8), C * H)], osem)

    def process(k, buf):
        @pl.when(k > 0)
        def _():
            out_copy(k).wait()

        def h_body(h, carry):
            off = h * LANES
            for i in range(C):
                base = i * F
                s = buf[base, pl.ds(off, LANES)]
                for f in range(1, F):
                    s = s + buf[base + f, pl.ds(off, LANES)]
                acc_v[pl.ds(i * H + off, LANES)] = s
            return carry

        lax.fori_loop(0, HV, h_body, 0)
        out_copy(k).start()

    gather(0, rows0, sem0).start()

    def pair_body(j, carry):
        k0 = 2 * j
        k1 = k0 + 1
        gather(k1, rows1, sem1).start()
        gather(k0, rows0, sem0).wait()
        process(k0, rows0)

        @pl.when(k1 + 1 < CHUNKS)
        def _():
            gather(k1 + 1, rows0, sem0).start()

        gather(k1, rows1, sem1).wait()
        process(k1, rows1)
        return carry

    lax.fori_loop(0, CHUNKS // 2, pair_body, 0)
    out_copy(CHUNKS - 1).wait()


_sc_call = pl.kernel(
    _body,
    out_type=jax.ShapeDtypeStruct((B * NB1 * H,), jnp.float32),
    mesh=plsc.VectorSubcoreMesh(core_axis_name="c", subcore_axis_name="s"),
    compiler_params=pltpu.CompilerParams(use_tc_tiling_on_sc=False),
    scratch_types=[
        pltpu.VMEM((NODES_PW * F,), jnp.int32),
        pltpu.VMEM((ROWS, H), jnp.float32),
        pltpu.VMEM((ROWS, H), jnp.float32),
        pltpu.VMEM((C * H,), jnp.float32),
        pltpu.VMEM((H,), jnp.float32),
        pltpu.SemaphoreType.DMA,
        pltpu.SemaphoreType.DMA,
        pltpu.SemaphoreType.DMA,
    ],
)


@jax.jit
def kernel(input_nodes, atom_encoder_weight, graph_token_weight):
    idx_flat = input_nodes.reshape(B * N * F)
    out = _sc_call(idx_flat, atom_encoder_weight, graph_token_weight.reshape(H))
    return out.reshape(B, NB1, H)


# speedup vs baseline: 2.4069x; 1.0015x over previous
import functools

import jax
import jax.numpy as jnp
from jax import lax
from jax.experimental import pallas as pl
from jax.experimental.pallas import tpu as pltpu
from jax.experimental.pallas import tpu_sc as plsc

B, N, F = 256, 128, 9
H = 768
NB1 = N + 1
NW = 32
BPW = B // NW
NODES_PW = BPW * N
C = 8
ROWS = C * F
CHUNKS_PB = N // C
CHUNKS = NODES_PW // C
LANES = 16
HV = H // LANES


def _body(idx_hbm, table_hbm, token_hbm, out_hbm,
          idx_v, rows0, rows1, acc_v, tok_v, sem0, sem1, osem):
    cid = lax.axis_index("c")
    sid = lax.axis_index("s")
    wid = sid * 2 + cid
    node0 = wid * NODES_PW

    idx_cp = pltpu.make_async_copy(
        idx_hbm.at[pl.ds(node0 * F, NODES_PW * F)], idx_v, sem0)
    idx_cp.start()
    pltpu.sync_copy(token_hbm, tok_v)
    tok_cps = []
    for b in range(BPW):
        row = (wid * BPW + b) * NB1
        cp = pltpu.make_async_copy(
            tok_v, out_hbm.at[pl.ds(pl.multiple_of(row * H, 8), H)], sem1)
        cp.start()
        tok_cps.append(cp)
    idx_cp.wait()
    for cp in tok_cps:
        cp.wait()

    def gather(k, buf, sem):
        start = pl.multiple_of(k * ROWS, 8)
        return pltpu.make_async_copy(
            table_hbm.at[idx_v.at[pl.ds(start, ROWS)]], buf, sem)

    def out_copy(k):
        b = k // CHUNKS_PB
        n = (k % CHUNKS_PB) * C
        row = (wid * BPW + b) * NB1 + 1 + n
        return pltpu.make_async_copy(
            acc_v, out_hbm.at[pl.ds(pl.multiple_of(row * H, 8), C * H)], osem)

    def process(k, buf):
        @pl.when(k > 0)
        def _():
            out_copy(k).wait()

        def h_body(h2, carry):
            for u in range(2):
                off = (2 * h2 + u) * LANES
                for i in range(C):
                    base = i * F
                    s = buf[base, pl.ds(off, LANES)]
                    for f in range(1, F):
                        s = s + buf[base + f, pl.ds(off, LANES)]
                    acc_v[pl.ds(i * H + off, LANES)] = s
            return carry

        lax.fori_loop(0, HV // 2, h_body, 0)
        out_copy(k).start()

    gather(0, rows0, sem0).start()

    def pair_body(j, carry):
        k0 = 2 * j
        k1 = k0 + 1
        gather(k1, rows1, sem1).start()
        gather(k0, rows0, sem0).wait()
        process(k0, rows0)

        @pl.when(k1 + 1 < CHUNKS)
        def _():
            gather(k1 + 1, rows0, sem0).start()

        gather(k1, rows1, sem1).wait()
        process(k1, rows1)
        return carry

    lax.fori_loop(0, CHUNKS // 2, pair_body, 0)
    out_copy(CHUNKS - 1).wait()


_sc_call = pl.kernel(
    _body,
    out_type=jax.ShapeDtypeStruct((B * NB1 * H,), jnp.float32),
    mesh=plsc.VectorSubcoreMesh(core_axis_name="c", subcore_axis_name="s"),
    compiler_params=pltpu.CompilerParams(use_tc_tiling_on_sc=False),
    scratch_types=[
        pltpu.VMEM((NODES_PW * F,), jnp.int32),
        pltpu.VMEM((ROWS, H), jnp.float32),
        pltpu.VMEM((ROWS, H), jnp.float32),
        pltpu.VMEM((C * H,), jnp.float32),
        pltpu.VMEM((H,), jnp.float32),
        pltpu.SemaphoreType.DMA,
        pltpu.SemaphoreType.DMA,
        pltpu.SemaphoreType.DMA,
    ],
)


@jax.jit
def kernel(input_nodes, atom_encoder_weight, graph_token_weight):
    idx_flat = input_nodes.reshape(B * N * F)
    out = _sc_call(idx_flat, atom_encoder_weight, graph_token_weight.reshape(H))
    return out.reshape(B, NB1, H)
